# baseline (speedup 1.0000x reference)
import jax
import jax.numpy as jnp
from jax.experimental import pallas as pl
from jax.experimental.pallas import tpu as pltpu

_DIM = 128
_SIZE = 262144
_NCHUNK = 8
_ROWS = _DIM // _NCHUNK


def _dma_body(src_hbm, dst_hbm, sems):
    for i in range(_NCHUNK):
        pltpu.make_async_copy(
            src_hbm.at[pl.ds(i * _ROWS, _ROWS)],
            dst_hbm.at[pl.ds(i * _ROWS, _ROWS)],
            sems.at[i],
        ).start()
    for i in range(_NCHUNK):
        pltpu.make_async_copy(
            src_hbm.at[pl.ds(i * _ROWS, _ROWS)],
            dst_hbm.at[pl.ds(i * _ROWS, _ROWS)],
            sems.at[i],
        ).wait()


def kernel(output, bank):
    bank_snapshot = pl.pallas_call(
        _dma_body,
        in_specs=[pl.BlockSpec(memory_space=pltpu.HBM)],
        out_specs=pl.BlockSpec(memory_space=pltpu.HBM),
        out_shape=jax.ShapeDtypeStruct((_DIM, _SIZE), jnp.float32),
        scratch_shapes=[pltpu.SemaphoreType.DMA((_NCHUNK,))],
    )(bank)
    return (output, bank_snapshot)


# speedup vs baseline: 33.4839x; 33.4839x over previous
import functools

import jax
import jax.numpy as jnp
from jax import lax
from jax.experimental import pallas as pl
from jax.experimental.pallas import tpu as pltpu
from jax.experimental.pallas import tpu_sc as plsc

_DIM = 128
_SIZE = 262144

_NC = 2
_NS = 16
_NW = _NC * _NS

_ROWS = 8
_NROWB = _DIM // _ROWS
_NCOLH = _NW // _NROWB
_CPW = _SIZE // _NCOLH
_CH = 4096
_NCHUNK = _CPW // _CH

_mesh = plsc.VectorSubcoreMesh(core_axis_name="c", subcore_axis_name="s")


@functools.partial(
    pl.kernel,
    mesh=_mesh,
    out_type=jax.ShapeDtypeStruct((_DIM, _SIZE), jnp.float32),
    scratch_types=[
        pltpu.VMEM((_ROWS, _CH), jnp.float32),
        pltpu.VMEM((_ROWS, _CH), jnp.float32),
        pltpu.SemaphoreType.DMA,
        pltpu.SemaphoreType.DMA,
        pltpu.SemaphoreType.DMA,
        pltpu.SemaphoreType.DMA,
    ],
)
def _sc_copy(bank_hbm, out_hbm, buf_a, buf_b, in_a, in_b, out_a, out_b):
    wid = lax.axis_index("s") * _NC + lax.axis_index("c")
    band = wid % _NROWB
    half = wid // _NROWB
    r0 = band * _ROWS
    c0 = half * _CPW

    bufs = (buf_a, buf_b)
    in_sems = (in_a, in_b)
    out_sems = (out_a, out_b)

    def _src(i):
        return bank_hbm.at[pl.ds(r0, _ROWS), pl.ds(c0 + i * _CH, _CH)]

    def _dst(i):
        return out_hbm.at[pl.ds(r0, _ROWS), pl.ds(c0 + i * _CH, _CH)]

    pltpu.make_async_copy(_src(0), bufs[0], in_sems[0]).start()
    for i in range(_NCHUNK):
        b = i % 2
        nb = (i + 1) % 2
        if i + 1 < _NCHUNK:
            if i >= 1:
                pltpu.make_async_copy(bufs[nb], _dst(i - 1), out_sems[nb]).wait()
            pltpu.make_async_copy(_src(i + 1), bufs[nb], in_sems[nb]).start()
        pltpu.make_async_copy(_src(i), bufs[b], in_sems[b]).wait()
        pltpu.make_async_copy(bufs[b], _dst(i), out_sems[b]).start()
    pltpu.make_async_copy(bufs[(_NCHUNK - 2) % 2], _dst(_NCHUNK - 2),
                          out_sems[(_NCHUNK - 2) % 2]).wait()
    pltpu.make_async_copy(bufs[(_NCHUNK - 1) % 2], _dst(_NCHUNK - 1),
                          out_sems[(_NCHUNK - 1) % 2]).wait()


def kernel(output, bank):
    return (output, _sc_copy(bank))
